# baseline (speedup 1.0000x reference)
import functools

import jax
import jax.numpy as jnp
from jax import lax
from jax.experimental import pallas as pl
from jax.experimental.pallas import tpu as pltpu
from jax.experimental.pallas import tpu_sc as plsc

_NUM_CORES = 2
_NUM_SUBCORES = 16
_NUM_WORKERS = _NUM_CORES * _NUM_SUBCORES

_EMBED_DIM = 64


def _make_gather(total_rows: int, dim: int):
    assert total_rows % _NUM_WORKERS == 0
    rows_per_worker = total_rows // _NUM_WORKERS
    chunk = 1664
    assert rows_per_worker % chunk == 0
    n_chunks = rows_per_worker // chunk

    mesh = plsc.VectorSubcoreMesh(
        core_axis_name="c",
        subcore_axis_name="s",
        num_cores=_NUM_CORES,
        num_subcores=_NUM_SUBCORES,
    )

    @functools.partial(
        pl.kernel,
        out_type=jax.ShapeDtypeStruct((total_rows, dim), jnp.float32),
        mesh=mesh,
        scratch_types=[
            pltpu.VMEM((rows_per_worker,), jnp.int32),
            pltpu.VMEM((chunk, dim), jnp.float32),
            pltpu.SemaphoreType.DMA,
        ],
        compiler_params=pltpu.CompilerParams(use_tc_tiling_on_sc=False),
    )
    def gather_kernel(table_hbm, idx_hbm, out_hbm, idx_v, rows_v, sem):
        wid = lax.axis_index("s") * _NUM_CORES + lax.axis_index("c")
        base = wid * rows_per_worker
        pltpu.sync_copy(idx_hbm.at[pl.ds(base, rows_per_worker)], idx_v)
        for j in range(n_chunks):
            pltpu.async_copy(
                table_hbm.at[idx_v.at[pl.ds(j * chunk, chunk)]], rows_v, sem
            ).wait()
            pltpu.sync_copy(rows_v, out_hbm.at[pl.ds(base + j * chunk, chunk)])

    return gather_kernel


def kernel(indices, weight):
    flat = indices.reshape(-1).astype(jnp.int32)
    total_rows = flat.shape[0]
    dim = weight.shape[1]
    out = _make_gather(total_rows, dim)(weight, flat)
    return out.reshape(indices.shape + (dim,))
